# baseline (speedup 1.0000x reference)
import functools

import jax
import jax.numpy as jnp
from jax.experimental import pallas as pl
from jax.experimental.pallas import tpu as pltpu

_TOKENS_PER_BLOCK = 4096


def _route_group(logits, num_experts):
    m = jnp.max(logits, axis=-1, keepdims=True)
    ex = jnp.exp(logits - m)
    denom = jnp.sum(ex, axis=-1, keepdims=True)
    recip = 1.0 / denom
    probs = ex * recip

    rows = logits.shape[0]
    eidx = jax.lax.broadcasted_iota(
        jnp.int32, (rows, num_experts), 1).astype(jnp.float32)
    big = float(num_experts)
    i1 = jnp.min(jnp.where(logits == m, eidx, big), axis=-1, keepdims=True)
    sel1 = eidx == i1
    m2 = jnp.max(jnp.where(sel1, -jnp.inf, logits), axis=-1, keepdims=True)
    i2 = jnp.min(jnp.where(jnp.logical_and(logits == m2, ~sel1), eidx, big),
                 axis=-1, keepdims=True)
    p1 = recip
    p2 = jnp.exp(m2 - m) * recip
    ssum = p1 + p2
    return probs, p1 / ssum, p2 / ssum, i1, i2


def _router_block(x_ref, w_ref, tkp_ref, tki_ref, probs_ref, esum_ref,
                  *, num_experts):
    h = w_ref.shape[0]
    w = w_ref[...]

    la = jnp.dot(x_ref[:, :h], w, preferred_element_type=jnp.float32)
    lb = jnp.dot(x_ref[:, h:], w, preferred_element_type=jnp.float32)
    probs_a, p1a, p2a, i1a, i2a = _route_group(la, num_experts)
    probs_b, p1b, p2b, i1b, i2b = _route_group(lb, num_experts)

    probs_ref[...] = jnp.concatenate([probs_a, probs_b], axis=-1)
    tkp_ref[...] = jnp.concatenate([p1a, p2a, p1b, p2b], axis=-1)
    tki_ref[...] = jnp.concatenate([i1a, i2a, i1b, i2b],
                                   axis=-1).astype(jnp.int32)

    esum_ref[...] = (jnp.sum(probs_a, axis=0, keepdims=True) +
                     jnp.sum(probs_b, axis=0, keepdims=True))[None]


def _aux_block(esum_ref, aux_ref, *, num_tokens):
    load = jnp.sum(esum_ref[...], axis=0) / num_tokens
    aux_ref[...] = jnp.sum(load * jnp.log(load + 1e-09), axis=-1,
                           keepdims=True)


def kernel(hidden_states, weight):
    b, s, h = hidden_states.shape
    e = weight.shape[1]
    n = b * s
    t = _TOKENS_PER_BLOCK
    num_blocks = n // t
    r = t // 2

    x = hidden_states.reshape(n // 2, 2 * h)

    body = functools.partial(_router_block, num_experts=e)

    tkp, tki, probs, esum = pl.pallas_call(
        body,
        grid=(num_blocks,),
        in_specs=[
            pl.BlockSpec((r, 2 * h), lambda i: (i, 0)),
            pl.BlockSpec((h, e), lambda i: (0, 0)),
        ],
        out_specs=[
            pl.BlockSpec((r, 4), lambda i: (i, 0)),
            pl.BlockSpec((r, 4), lambda i: (i, 0)),
            pl.BlockSpec((r, 2 * e), lambda i: (i, 0)),
            pl.BlockSpec((1, 1, e), lambda i: (i, 0, 0)),
        ],
        out_shape=[
            jax.ShapeDtypeStruct((n // 2, 4), jnp.float32),
            jax.ShapeDtypeStruct((n // 2, 4), jnp.int32),
            jax.ShapeDtypeStruct((n // 2, 2 * e), jnp.float32),
            jax.ShapeDtypeStruct((num_blocks, 1, e), jnp.float32),
        ],
        compiler_params=pltpu.CompilerParams(
            dimension_semantics=("arbitrary",)),
    )(x, weight)

    aux = pl.pallas_call(
        functools.partial(_aux_block, num_tokens=n),
        out_shape=jax.ShapeDtypeStruct((1, 1), jnp.float32),
    )(esum)

    top_k_probs = tkp.reshape(b, s, 2)
    top_k_indices = tki.reshape(b, s, 2)
    routing_probs = probs.reshape(b, s, e)
    aux_loss = aux[0, 0]
    expert_counts = jnp.zeros((e,), dtype=jnp.int32)
    return (top_k_probs, top_k_indices, aux_loss, expert_counts, routing_probs)


# speedup vs baseline: 2.0549x; 2.0549x over previous
import functools

import jax
import jax.numpy as jnp
from jax.experimental import pallas as pl
from jax.experimental.pallas import tpu as pltpu

_TOKENS_PER_BLOCK = 2048


def _router_block(x_ref, w_ref, tkp_ref, tki_ref, probs_ref, esum_ref,
                  *, num_experts):
    t = x_ref.shape[0]

    logits = jnp.dot(x_ref[...], w_ref[...], preferred_element_type=jnp.float32)

    m = jnp.max(logits, axis=-1, keepdims=True)
    ex = jnp.exp(logits - m)
    denom = jnp.sum(ex, axis=-1, keepdims=True)
    recip = 1.0 / denom
    probs = ex * recip
    probs_ref[...] = jnp.concatenate([probs, probs], axis=-1)

    eidx = jax.lax.broadcasted_iota(
        jnp.int32, (t, num_experts), 1).astype(jnp.float32)
    big = float(num_experts)
    i1 = jnp.min(jnp.where(logits == m, eidx, big), axis=-1, keepdims=True)
    sel1 = eidx == i1
    m2 = jnp.max(jnp.where(sel1, -jnp.inf, logits), axis=-1, keepdims=True)
    i2 = jnp.min(jnp.where(jnp.logical_and(logits == m2, ~sel1), eidx, big),
                 axis=-1, keepdims=True)
    p1 = recip
    p2 = jnp.exp(m2 - m) * recip

    ssum = p1 + p2
    tkp_ref[...] = jnp.concatenate([p1 / ssum, p2 / ssum], axis=-1)
    tki_ref[...] = jnp.concatenate([i1, i2], axis=-1).astype(jnp.int32)

    esum_ref[...] = jnp.sum(probs, axis=0, keepdims=True)[None]


def _aux_block(esum_ref, aux_ref, *, num_tokens):
    load = jnp.sum(esum_ref[...], axis=0) / num_tokens
    aux_ref[...] = jnp.sum(load * jnp.log(load + 1e-09), axis=-1,
                           keepdims=True)


def kernel(hidden_states, weight):
    b, s, h = hidden_states.shape
    e = weight.shape[1]
    n = b * s
    t = _TOKENS_PER_BLOCK
    num_blocks = n // t

    x = hidden_states.reshape(n, h)

    body = functools.partial(_router_block, num_experts=e)

    tkp, tki, probs, esum = pl.pallas_call(
        body,
        grid=(num_blocks,),
        in_specs=[
            pl.BlockSpec((t, h), lambda i: (i, 0)),
            pl.BlockSpec((h, e), lambda i: (0, 0)),
        ],
        out_specs=[
            pl.BlockSpec((t, 2), lambda i: (i, 0)),
            pl.BlockSpec((t, 2), lambda i: (i, 0)),
            pl.BlockSpec((t, 2 * e), lambda i: (i, 0)),
            pl.BlockSpec((1, 1, e), lambda i: (i, 0, 0)),
        ],
        out_shape=[
            jax.ShapeDtypeStruct((n, 2), jnp.float32),
            jax.ShapeDtypeStruct((n, 2), jnp.int32),
            jax.ShapeDtypeStruct((n, 2 * e), jnp.float32),
            jax.ShapeDtypeStruct((num_blocks, 1, e), jnp.float32),
        ],
        compiler_params=pltpu.CompilerParams(
            dimension_semantics=("parallel",)),
    )(x, weight)

    aux = pl.pallas_call(
        functools.partial(_aux_block, num_tokens=n),
        out_shape=jax.ShapeDtypeStruct((1, 1), jnp.float32),
    )(esum)

    top_k_probs = tkp.reshape(b, s, 2)
    top_k_indices = tki.reshape(b, s, 2)
    routing_probs = probs[:, :e].reshape(b, s, e)
    aux_loss = aux[0, 0]
    expert_counts = jnp.zeros((e,), dtype=jnp.int32)
    return (top_k_probs, top_k_indices, aux_loss, expert_counts, routing_probs)


# speedup vs baseline: 2.7035x; 1.3156x over previous
import functools

import jax
import jax.numpy as jnp
from jax.experimental import pallas as pl
from jax.experimental.pallas import tpu as pltpu

_TOKENS_PER_BLOCK = 4096


def _router_block(x_ref, w_ref, tkp_ref, tki_ref, probs_ref, aux_ref,
                  load_acc, *, num_blocks, num_tokens, num_experts):
    i = pl.program_id(0)
    t = x_ref.shape[0]

    logits = jnp.dot(x_ref[...], w_ref[...], preferred_element_type=jnp.float32)

    m = jnp.max(logits, axis=-1, keepdims=True)
    ex = jnp.exp(logits - m)
    denom = jnp.sum(ex, axis=-1, keepdims=True)
    recip = 1.0 / denom
    probs = ex * recip
    probs_ref[...] = probs

    eidx = jax.lax.broadcasted_iota(
        jnp.int32, (t, num_experts), 1).astype(jnp.float32)
    big = float(num_experts)
    i1 = jnp.min(jnp.where(logits == m, eidx, big), axis=-1, keepdims=True)
    sel1 = eidx == i1
    m2 = jnp.max(jnp.where(sel1, -jnp.inf, logits), axis=-1, keepdims=True)
    i2 = jnp.min(jnp.where(jnp.logical_and(logits == m2, ~sel1), eidx, big),
                 axis=-1, keepdims=True)
    p1 = recip
    p2 = jnp.exp(m2 - m) * recip

    ssum = p1 + p2
    tkp_ref[...] = jnp.concatenate([p1 / ssum, p2 / ssum], axis=-1)
    tki_ref[...] = jnp.concatenate([i1, i2], axis=-1).astype(jnp.int32)

    block_sum = jnp.sum(probs, axis=0, keepdims=True)

    @pl.when(i == 0)
    def _():
        load_acc[...] = block_sum

    @pl.when(i > 0)
    def _():
        load_acc[...] = load_acc[...] + block_sum

    @pl.when(i == num_blocks - 1)
    def _():
        load = load_acc[...] / num_tokens
        aux_ref[...] = jnp.sum(load * jnp.log(load + 1e-09), axis=-1,
                               keepdims=True)


def kernel(hidden_states, weight):
    b, s, h = hidden_states.shape
    e = weight.shape[1]
    n = b * s
    t = _TOKENS_PER_BLOCK
    num_blocks = n // t

    x = hidden_states.reshape(n, h)

    body = functools.partial(_router_block, num_blocks=num_blocks,
                             num_tokens=n, num_experts=e)

    tkp, tki, probs, aux = pl.pallas_call(
        body,
        grid=(num_blocks,),
        in_specs=[
            pl.BlockSpec((t, h), lambda i: (i, 0)),
            pl.BlockSpec((h, e), lambda i: (0, 0)),
        ],
        out_specs=[
            pl.BlockSpec((t, 2), lambda i: (i, 0)),
            pl.BlockSpec((t, 2), lambda i: (i, 0)),
            pl.BlockSpec((t, e), lambda i: (i, 0)),
            pl.BlockSpec((1, 1), lambda i: (0, 0)),
        ],
        out_shape=[
            jax.ShapeDtypeStruct((n, 2), jnp.float32),
            jax.ShapeDtypeStruct((n, 2), jnp.int32),
            jax.ShapeDtypeStruct((n, e), jnp.float32),
            jax.ShapeDtypeStruct((1, 1), jnp.float32),
        ],
        scratch_shapes=[pltpu.VMEM((1, e), jnp.float32)],
    )(x, weight)

    top_k_probs = tkp.reshape(b, s, 2)
    top_k_indices = tki.reshape(b, s, 2)
    routing_probs = probs.reshape(b, s, e)
    aux_loss = aux[0, 0]
    expert_counts = jnp.zeros((e,), dtype=jnp.int32)
    return (top_k_probs, top_k_indices, aux_loss, expert_counts, routing_probs)


# speedup vs baseline: 2.7211x; 1.0065x over previous
import functools

import jax
import jax.numpy as jnp
from jax.experimental import pallas as pl
from jax.experimental.pallas import tpu as pltpu

_TOKENS_PER_BLOCK = 4096


def _router_block(x_ref, w_ref, tkp_ref, tki_ref, probs_hbm, aux_ref,
                  load_acc, pbuf, psem, *, num_blocks, num_tokens,
                  num_experts):
    i = pl.program_id(0)
    t = x_ref.shape[0]
    slot = jax.lax.rem(i, 2)

    @pl.when(i >= 2)
    def _():
        pltpu.make_async_copy(
            pbuf.at[slot],
            probs_hbm.at[pl.ds((i - 2) * t, t), :],
            psem.at[slot]).wait()

    logits = jnp.dot(x_ref[...], w_ref[...], preferred_element_type=jnp.float32)

    m = jnp.max(logits, axis=-1, keepdims=True)
    ex = jnp.exp(logits - m)
    denom = jnp.sum(ex, axis=-1, keepdims=True)
    recip = 1.0 / denom
    probs = ex * recip
    pbuf[slot] = probs
    pltpu.make_async_copy(
        pbuf.at[slot],
        probs_hbm.at[pl.ds(i * t, t), :],
        psem.at[slot]).start()

    eidx = jax.lax.broadcasted_iota(
        jnp.int32, (t, num_experts), 1).astype(jnp.float32)
    big = float(num_experts)
    i1 = jnp.min(jnp.where(logits == m, eidx, big), axis=-1, keepdims=True)
    sel1 = eidx == i1
    m2 = jnp.max(jnp.where(sel1, -jnp.inf, logits), axis=-1, keepdims=True)
    i2 = jnp.min(jnp.where(jnp.logical_and(logits == m2, ~sel1), eidx, big),
                 axis=-1, keepdims=True)
    p1 = recip
    p2 = jnp.exp(m2 - m) * recip

    ssum = p1 + p2
    tkp_ref[...] = jnp.concatenate([p1 / ssum, p2 / ssum], axis=-1)
    tki_ref[...] = jnp.concatenate([i1, i2], axis=-1).astype(jnp.int32)

    block_sum = jnp.sum(probs, axis=0, keepdims=True)

    @pl.when(i == 0)
    def _():
        load_acc[...] = block_sum

    @pl.when(i > 0)
    def _():
        load_acc[...] = load_acc[...] + block_sum

    @pl.when(i == num_blocks - 1)
    def _():
        load = load_acc[...] / num_tokens
        aux_ref[...] = jnp.sum(load * jnp.log(load + 1e-09), axis=-1,
                               keepdims=True)
        other = 1 - slot
        pltpu.make_async_copy(
            pbuf.at[other],
            probs_hbm.at[pl.ds((num_blocks - 2) * t, t), :],
            psem.at[other]).wait()
        pltpu.make_async_copy(
            pbuf.at[slot],
            probs_hbm.at[pl.ds((num_blocks - 1) * t, t), :],
            psem.at[slot]).wait()


def kernel(hidden_states, weight):
    b, s, h = hidden_states.shape
    e = weight.shape[1]
    n = b * s
    t = _TOKENS_PER_BLOCK
    num_blocks = n // t

    x = hidden_states.reshape(n, h)

    body = functools.partial(_router_block, num_blocks=num_blocks,
                             num_tokens=n, num_experts=e)

    tkp, tki, probs, aux = pl.pallas_call(
        body,
        grid=(num_blocks,),
        in_specs=[
            pl.BlockSpec((t, h), lambda i: (i, 0)),
            pl.BlockSpec((h, e), lambda i: (0, 0)),
        ],
        out_specs=[
            pl.BlockSpec((t, 2), lambda i: (i, 0)),
            pl.BlockSpec((t, 2), lambda i: (i, 0)),
            pl.BlockSpec(memory_space=pl.ANY),
            pl.BlockSpec((1, 1), lambda i: (0, 0)),
        ],
        out_shape=[
            jax.ShapeDtypeStruct((n, 2), jnp.float32),
            jax.ShapeDtypeStruct((n, 2), jnp.int32),
            jax.ShapeDtypeStruct((n, e), jnp.float32),
            jax.ShapeDtypeStruct((1, 1), jnp.float32),
        ],
        scratch_shapes=[pltpu.VMEM((1, e), jnp.float32),
                        pltpu.VMEM((2, t, e), jnp.float32),
                        pltpu.SemaphoreType.DMA((2,))],
    )(x, weight)

    top_k_probs = tkp.reshape(b, s, 2)
    top_k_indices = tki.reshape(b, s, 2)
    routing_probs = probs.reshape(b, s, e)
    aux_loss = aux[0, 0]
    expert_counts = jnp.zeros((e,), dtype=jnp.int32)
    return (top_k_probs, top_k_indices, aux_loss, expert_counts, routing_probs)
